# Initial kernel scaffold; baseline (speedup 1.0000x reference)
import jax
import jax.numpy as jnp
from jax.experimental import pallas as pl


def kernel(x, base_adj, base_edge_index, W_in, b_in, g_in, be_in, W_h1, b_h1, g_h1, be_h1, W_h2, b_h2, g_h2, be_h2):
    raise NotImplementedError("write your pallas kernel here")


# speedup vs baseline: 58.6344x; 58.6344x over previous
import functools

import jax
import jax.numpy as jnp
from jax import lax
from jax.experimental import pallas as pl
from jax.experimental.pallas import tpu as pltpu
from jax.experimental.pallas import tpu_sc as plsc

N = 2000
ROWS_PER_CORE = 1000
E = 32000
NTILES = 16
EPT = E // NTILES
ROWS_P = (504, 496)
DUMP_BASE = ROWS_P[0] * N
ZSPAN = 63008
SHARED_LEN = 16 * ZSPAN
DRAIN_P = (ROWS_P[0] * N // NTILES, ROWS_P[1] * N // NTILES)


def _build_cnt(ei):
    mesh = plsc.VectorSubcoreMesh(core_axis_name="c", subcore_axis_name="s")

    @functools.partial(
        pl.kernel,
        mesh=mesh,
        out_type=jax.ShapeDtypeStruct((N * N,), jnp.float32),
        scratch_types=[
            pltpu.VMEM((EPT,), jnp.int32),
            pltpu.VMEM((EPT,), jnp.int32),
            pltpu.VMEM((16, 128), jnp.int32),
            pltpu.VMEM((128,), jnp.float32),
            pltpu.VMEM((8192,), jnp.float32),
            pltpu.VMEM((25000,), jnp.float32),
            pltpu.VMEM_SHARED((SHARED_LEN,), jnp.float32),
        ],
    )
    def k(ei_hbm, cnt_hbm, src_v, dst_v, idx_v, ones_v, zero_v, stage_v, shared):
        c = lax.axis_index("c")
        s = lax.axis_index("s")
        dump = DUMP_BASE + s

        eoff = pl.multiple_of(s * EPT, 8)
        pltpu.sync_copy(ei_hbm.at[pl.ds(eoff, EPT)], src_v)
        doff0 = pl.multiple_of(E + s * EPT, 8)
        pltpu.sync_copy(ei_hbm.at[pl.ds(doff0, EPT)], dst_v)

        def zbody(i, carry):
            zero_v[pl.ds(i * 16, 16)] = jnp.zeros((16,), jnp.float32)
            return carry

        lax.fori_loop(0, 512, zbody, 0)
        for kk in range(8):
            ones_v[pl.ds(kk * 16, 16)] = jnp.ones((16,), jnp.float32)

        for p in range(2):
            rows = ROWS_P[p]
            lo = c * ROWS_PER_CORE + p * ROWS_P[0]

            if p > 0:
                plsc.subcore_barrier()

            zbase = s * ZSPAN
            for j in range(7):
                zoff = pl.multiple_of(zbase + j * 8192, 8)
                pltpu.sync_copy(zero_v, shared.at[pl.ds(zoff, 8192)])
            rem = ZSPAN - 7 * 8192
            zoff = pl.multiple_of(zbase + 7 * 8192, 8)
            pltpu.sync_copy(zero_v.at[pl.ds(0, rem)], shared.at[pl.ds(zoff, rem)])

            def chunk(j, kk):
                off = (j * 8 + kk) * 16
                d = dst_v[pl.ds(off, 16)]
                sv = src_v[pl.ds(off, 16)]
                inr = (d >= lo) & (d < lo + rows)
                flat = (d - lo) * N + sv
                idx_v[j, pl.ds(kk * 16, 16)] = jnp.where(inr, flat, dump)

            def jbody(j, carry):
                for kk in range(8):
                    chunk(j, kk)
                return carry

            lax.fori_loop(0, 15, jbody, 0)
            for kk in range(5):
                chunk(15, kk)
            for kk in range(5, 8):
                idx_v[15, pl.ds(kk * 16, 16)] = jnp.zeros((16,), jnp.int32) + dump

            plsc.subcore_barrier()

            for j in range(16):
                pltpu.sync_copy(ones_v, shared.at[idx_v.at[j]], add=True)

            plsc.subcore_barrier()

            span = DRAIN_P[p]
            hbm_base = (c * ROWS_PER_CORE + p * ROWS_P[0]) * N
            for off in range(0, span, 25000):
                ln = min(25000, span - off)
                soff = pl.multiple_of(s * span + off, 8)
                doff = pl.multiple_of(hbm_base + s * span + off, 8)
                pltpu.sync_copy(shared.at[pl.ds(soff, ln)], stage_v.at[pl.ds(0, ln)])
                pltpu.sync_copy(stage_v.at[pl.ds(0, ln)], cnt_hbm.at[pl.ds(doff, ln)])

    return k(ei.reshape(2 * E))


def _gcn_body(cnt_ref, x_ref, wi, bi, gi, bei, w1, b1, g1, be1, w2, b2, g2, be2,
              o_ref, dinv_ref):
    g = pl.program_id(0)

    @pl.when(g == 0)
    def _():
        deg = jnp.sum(cnt_ref[...], axis=1, keepdims=True) + 1.0
        dinv_ref[...] = lax.rsqrt(jnp.maximum(deg, 1e-12))

    dinv = dinv_ref[...]
    A = cnt_ref[...]
    xb = x_ref[0]

    def gcn(inp, W, b):
        h = jnp.dot(inp, W[...], preferred_element_type=jnp.float32)
        z = jnp.dot(A, h * dinv, preferred_element_type=jnp.float32)
        return (z + dinv * h) * dinv + b[...]

    def lnrelu(z, gamma, beta):
        mu = jnp.mean(z, axis=-1, keepdims=True)
        zc = z - mu
        var = jnp.mean(zc * zc, axis=-1, keepdims=True)
        return jnp.maximum(zc * lax.rsqrt(var + 1e-5) * gamma[...] + beta[...], 0.0)

    z = gcn(xb, wi, bi)
    y = lnrelu(z, gi, bei)
    z = gcn(y, w1, b1)
    y = lnrelu(z, g1, be1) + y
    z = gcn(y, w2, b2)
    y = lnrelu(z, g2, be2) + y
    o_ref[0] = y


def _gcn_tc(cnt, xg, W_in, b_in, g_in, be_in, W_h1, b_h1, g_h1, be_h1,
            W_h2, b_h2, g_h2, be_h2, interpret=False):
    G, n, C = xg.shape
    H = W_in.shape[1]
    full = lambda g: (0, 0)
    vec = pl.BlockSpec((1, H), full)
    return pl.pallas_call(
        _gcn_body,
        grid=(G,),
        in_specs=[
            pl.BlockSpec((n, n), full),
            pl.BlockSpec((1, n, C), lambda g: (g, 0, 0)),
            pl.BlockSpec((C, H), full), vec, vec, vec,
            pl.BlockSpec((H, H), full), vec, vec, vec,
            pl.BlockSpec((H, H), full), vec, vec, vec,
        ],
        out_specs=pl.BlockSpec((1, n, H), lambda g: (g, 0, 0)),
        out_shape=jax.ShapeDtypeStruct((G, n, H), jnp.float32),
        scratch_shapes=[pltpu.VMEM((n, 1), jnp.float32)],
        interpret=interpret,
    )(cnt, xg,
      W_in, b_in.reshape(1, H), g_in.reshape(1, H), be_in.reshape(1, H),
      W_h1, b_h1.reshape(1, H), g_h1.reshape(1, H), be_h1.reshape(1, H),
      W_h2, b_h2.reshape(1, H), g_h2.reshape(1, H), be_h2.reshape(1, H))


def kernel(x, base_adj, base_edge_index, W_in, b_in, g_in, be_in,
           W_h1, b_h1, g_h1, be_h1, W_h2, b_h2, g_h2, be_h2):
    Bx, Tx, Nx, Cx = x.shape
    G = Bx * Tx
    H = W_in.shape[1]
    cnt = _build_cnt(base_edge_index).reshape(Nx, Nx)
    xg = x.reshape(G, Nx, Cx)
    out = _gcn_tc(cnt, xg, W_in, b_in, g_in, be_in, W_h1, b_h1, g_h1, be_h1,
                  W_h2, b_h2, g_h2, be_h2)
    return out.reshape(Bx, Tx, Nx, H)
